# baseline (speedup 1.0000x reference)
import functools

import jax
import jax.numpy as jnp
from jax import lax
from jax.experimental import pallas as pl
from jax.experimental.pallas import tpu as pltpu
from jax.experimental.pallas import tpu_sc as plsc

B = 1024
SEQ = 800
HID = 64
SPLIT = 200
NUM_WORKERS = 32
ROWS_PER_WORKER = B // NUM_WORKERS
HALFSEQ = SEQ // 2
N_CHUNKS = 5
CHUNK = HALFSEQ // N_CHUNKS


def kernel(codes, W0, W1):
    codes = codes.astype(jnp.int32)
    W01 = jnp.concatenate([W0, W1], axis=1)
    mesh = plsc.VectorSubcoreMesh(core_axis_name="c", subcore_axis_name="s")

    @functools.partial(
        pl.kernel,
        mesh=mesh,
        out_type=jax.ShapeDtypeStruct((B, SEQ, HID), jnp.float32),
        compiler_params=pltpu.CompilerParams(use_tc_tiling_on_sc=False),
        scratch_types=[
            pltpu.VMEM((HALFSEQ,), jnp.int32),
            pltpu.VMEM((HALFSEQ,), jnp.int32),
            pltpu.VMEM((HALFSEQ, 2 * HID), jnp.float32),
            pltpu.VMEM((HALFSEQ, 2 * HID), jnp.float32),
            pltpu.SemaphoreType.DMA,
            pltpu.SemaphoreType.DMA,
            pltpu.SemaphoreType.DMA,
            pltpu.SemaphoreType.DMA,
            pltpu.SemaphoreType.DMA,
            pltpu.SemaphoreType.DMA,
        ],
    )
    def run(codes_hbm, w_hbm, out_hbm,
            idx_a, idx_b, buf_a, buf_b,
            sem_ga, sem_gb, sem_oa, sem_ob, sem_ia, sem_ib):
        wid = lax.axis_index("s") * 2 + lax.axis_index("c")
        b0 = wid * ROWS_PER_WORKER
        zeros16 = jnp.zeros((16,), jnp.float32)

        def fire_idx(b, h, idx_ref, sem):
            pltpu.async_copy(codes_hbm.at[b, pl.ds(h * HALFSEQ, HALFSEQ)], idx_ref, sem)

        def fire_gathers(idx_ref, buf_ref, sem):
            for c in range(N_CHUNKS):
                start = c * CHUNK
                pltpu.async_copy(
                    w_hbm.at[idx_ref.at[pl.ds(start, CHUNK)]],
                    buf_ref.at[pl.ds(start, CHUNK)],
                    sem,
                )

        def fire_out_h0(b, buf_ref, sem):
            pltpu.async_copy(
                buf_ref.at[pl.ds(0, SPLIT), pl.ds(0, HID)],
                out_hbm.at[b, pl.ds(0, SPLIT)],
                sem,
            )
            pltpu.async_copy(
                buf_ref.at[pl.ds(SPLIT, HALFSEQ - SPLIT), pl.ds(HID, HID)],
                out_hbm.at[b, pl.ds(SPLIT, HALFSEQ - SPLIT)],
                sem,
            )

        def fire_out_h1(b, buf_ref, sem):
            pltpu.async_copy(
                buf_ref.at[:, pl.ds(HID, HID)],
                out_hbm.at[b, pl.ds(HALFSEQ, HALFSEQ)],
                sem,
            )

        def drain(dummy_src, dst_ref, sem):
            pltpu.make_async_copy(dummy_src, dst_ref, sem).wait()

        def drain_idx(sem, idx_ref):
            drain(codes_hbm.at[b0, pl.ds(0, HALFSEQ)], idx_ref, sem)

        def drain_gathers(sem, buf_ref):
            drain(w_hbm.at[pl.ds(0, HALFSEQ)], buf_ref, sem)

        def drain_out(sem, buf_ref):
            drain(out_hbm.at[b0, pl.ds(0, HALFSEQ)], buf_ref.at[:, pl.ds(0, HID)], sem)

        def scan_half(idx_ref):
            def sb(ci, acc):
                return jnp.minimum(acc, idx_ref[pl.ds(ci * 16, 16)])

            accv = lax.fori_loop(0, HALFSEQ // 16, sb, jnp.full((16,), 1, jnp.int32))
            mn = accv[0]
            for j in range(1, 16):
                mn = jnp.minimum(mn, accv[j])
            return mn

        def fix_half(idx_ref, buf_ref):
            def fb(ci, c2):
                idx16 = idx_ref[pl.ds(ci * 16, 16)]
                for j in range(16):
                    @pl.when(idx16[j] == 0)
                    def _():
                        for k in range(2 * HID // 16):
                            buf_ref[ci * 16 + j, pl.ds(k * 16, 16)] = zeros16

                return c2

            lax.fori_loop(0, HALFSEQ // 16, fb, 0)

        pltpu.sync_copy(codes_hbm.at[b0, pl.ds(0, HALFSEQ)], idx_a)
        fire_gathers(idx_a, buf_a, sem_ga)
        fire_idx(b0, 1, idx_b, sem_ib)

        def body(g, carry):
            b = b0 + g

            mna = scan_half(idx_a)
            drain_idx(sem_ib, idx_b)

            @pl.when(g > 0)
            def _():
                drain_out(sem_ob, buf_b)

            fire_gathers(idx_b, buf_b, sem_gb)
            drain_gathers(sem_ga, buf_a)

            @pl.when(mna == 0)
            def _():
                fix_half(idx_a, buf_a)

            @pl.when(g < ROWS_PER_WORKER - 1)
            def _():
                fire_idx(b + 1, 0, idx_a, sem_ia)

            fire_out_h0(b, buf_a, sem_oa)

            mnb = scan_half(idx_b)

            @pl.when(g < ROWS_PER_WORKER - 1)
            def _():
                drain_idx(sem_ia, idx_a)
                drain_out(sem_oa, buf_a)
                fire_gathers(idx_a, buf_a, sem_ga)

            drain_gathers(sem_gb, buf_b)

            @pl.when(mnb == 0)
            def _():
                fix_half(idx_b, buf_b)

            @pl.when(g < ROWS_PER_WORKER - 1)
            def _():
                fire_idx(b + 1, 1, idx_b, sem_ib)

            fire_out_h1(b, buf_b, sem_ob)
            return carry

        lax.fori_loop(0, ROWS_PER_WORKER, body, 0)

        drain_out(sem_oa, buf_a)
        drain_out(sem_ob, buf_b)

    return run(codes, W01)


# speedup vs baseline: 1.0762x; 1.0762x over previous
import functools

import jax
import jax.numpy as jnp
from jax import lax
from jax.experimental import pallas as pl
from jax.experimental.pallas import tpu as pltpu
from jax.experimental.pallas import tpu_sc as plsc

B = 1024
SEQ = 800
HID = 64
SPLIT = 200
NUM_WORKERS = 32
ROWS_PER_WORKER = B // NUM_WORKERS
HALF = ROWS_PER_WORKER // 2
CHUNKS = ((0, 128), (128, 72), (200, 128), (328, 128), (456, 128), (584, 128), (712, 88))


def _sc_gather(codes, W0, W1):
    mesh = plsc.VectorSubcoreMesh(core_axis_name="c", subcore_axis_name="s")

    @functools.partial(
        pl.kernel,
        mesh=mesh,
        out_type=jax.ShapeDtypeStruct((B, SEQ, HID), jnp.float32),
        compiler_params=pltpu.CompilerParams(use_tc_tiling_on_sc=False),
        scratch_types=[
            pltpu.VMEM((SEQ,), jnp.int32),
            pltpu.VMEM((SEQ,), jnp.int32),
            pltpu.VMEM((SEQ, HID), jnp.float32),
            pltpu.VMEM((SEQ, HID), jnp.float32),
            pltpu.SemaphoreType.DMA,
            pltpu.SemaphoreType.DMA,
            pltpu.SemaphoreType.DMA,
            pltpu.SemaphoreType.DMA,
            pltpu.SemaphoreType.DMA,
            pltpu.SemaphoreType.DMA,
        ],
    )
    def run(codes_hbm, w0_hbm, w1_hbm, out_hbm,
            idx_a, idx_b, buf_a, buf_b,
            sem_ga, sem_gb, sem_oa, sem_ob, sem_ia, sem_ib):
        wid = lax.axis_index("s") * 2 + lax.axis_index("c")
        b0 = wid * ROWS_PER_WORKER
        zeros16 = jnp.zeros((16,), jnp.float32)

        def fire_gathers(idx_ref, buf_ref, sem):
            for start, ln in CHUNKS:
                tbl = w0_hbm if start < SPLIT else w1_hbm
                pltpu.async_copy(
                    tbl.at[idx_ref.at[pl.ds(start, ln)]],
                    buf_ref.at[pl.ds(start, ln)],
                    sem,
                )

        def drain(dummy_src, dst_ref, sem):
            pltpu.make_async_copy(dummy_src, dst_ref, sem).wait()

        def scan_row(idx_ref):
            def sb(ci, acc):
                return jnp.minimum(acc, idx_ref[pl.ds(ci * 16, 16)])

            accv = lax.fori_loop(0, SEQ // 16, sb, jnp.full((16,), 1, jnp.int32))
            mn = accv[0]
            for j in range(1, 16):
                mn = jnp.minimum(mn, accv[j])
            return mn

        def fix_row(idx_ref, buf_ref):
            def fb(ci, c2):
                idx16 = idx_ref[pl.ds(ci * 16, 16)]
                for j in range(16):
                    @pl.when(idx16[j] == 0)
                    def _():
                        for k in range(HID // 16):
                            buf_ref[ci * 16 + j, pl.ds(k * 16, 16)] = zeros16

                return c2

            lax.fori_loop(0, SEQ // 16, fb, 0)

        pltpu.sync_copy(codes_hbm.at[b0], idx_a)
        fire_gathers(idx_a, buf_a, sem_ga)
        pltpu.async_copy(codes_hbm.at[b0 + 1], idx_b, sem_ib)

        def body(g, carry):
            ra = b0 + 2 * g

            mna = scan_row(idx_a)
            drain(codes_hbm.at[b0], idx_b, sem_ib)

            @pl.when(g > 0)
            def _():
                drain(out_hbm.at[b0], buf_b, sem_ob)

            fire_gathers(idx_b, buf_b, sem_gb)
            drain(out_hbm.at[b0], buf_a, sem_ga)

            @pl.when(mna == 0)
            def _():
                fix_row(idx_a, buf_a)

            @pl.when(g < HALF - 1)
            def _():
                pltpu.async_copy(codes_hbm.at[ra + 2], idx_a, sem_ia)

            pltpu.async_copy(buf_a, out_hbm.at[ra], sem_oa)

            mnb = scan_row(idx_b)

            @pl.when(g < HALF - 1)
            def _():
                drain(codes_hbm.at[b0], idx_a, sem_ia)
                drain(out_hbm.at[b0], buf_a, sem_oa)
                fire_gathers(idx_a, buf_a, sem_ga)

            drain(out_hbm.at[b0], buf_b, sem_gb)

            @pl.when(mnb == 0)
            def _():
                fix_row(idx_b, buf_b)

            @pl.when(g < HALF - 1)
            def _():
                pltpu.async_copy(codes_hbm.at[ra + 3], idx_b, sem_ib)

            pltpu.async_copy(buf_b, out_hbm.at[ra + 1], sem_ob)
            return carry

        lax.fori_loop(0, HALF, body, 0)

        drain(out_hbm.at[b0], buf_a, sem_oa)
        drain(out_hbm.at[b0], buf_b, sem_ob)

    return run(codes, W0, W1)


def kernel(codes, W0, W1):
    codes = codes.astype(jnp.int32)
    return _sc_gather(codes, W0, W1)
